# baseline (speedup 1.0000x reference)
import jax
import jax.numpy as jnp
from jax.experimental import pallas as pl
from jax.experimental.pallas import tpu as pltpu

_PATCH = 14
_NPOS = _PATCH * _PATCH
_NPAD = 200
_NLEV = 256
_IB = 16


def _encoder_body(x_ref, pw_ref, lw_ref, out_ref, scr_ref):
    iota_bf = jax.lax.broadcasted_iota(
        jnp.int32, (_NPAD, _NLEV), 1).astype(jnp.bfloat16)
    for i in range(_IB):
        idx = jnp.round(x_ref[i] * (_NLEV - 1.0)).astype(jnp.bfloat16)
        cnt = (idx[0][:, None] == iota_bf).astype(jnp.bfloat16)
        for p in range(1, 4):
            cnt += (idx[p][:, None] == iota_bf).astype(jnp.bfloat16)
        scr_ref[i * _NPAD:(i + 1) * _NPAD, :] = cnt
    m = jax.lax.dot_general(
        scr_ref[...], lw_ref[...], (((1,), (0,)), ((), ())),
        preferred_element_type=jnp.float32,
    )
    pw = pw_ref[...]
    for i in range(_IB):
        s = jnp.sum(m[i * _NPAD:(i + 1) * _NPAD] * pw, axis=0)
        out_ref[i, :] = jnp.where(s > 0.0, 1.0, -1.0)


def kernel(x, position_weight, level_weight):
    B, C, H, W = x.shape
    p = _PATCH
    D = position_weight.shape[1]
    x_pj = x.reshape(B, C, H // p, p, W // p, p)
    x_pj = x_pj.transpose(0, 1, 2, 4, 3, 5).reshape(B, 4, p * p)
    x_pj = jnp.pad(x_pj, ((0, 0), (0, 0), (0, _NPAD - _NPOS)),
                   constant_values=-1.0)
    lw_bf16 = level_weight.astype(jnp.bfloat16)
    pw_pad = jnp.pad(position_weight, ((0, _NPAD - _NPOS), (0, 0)))

    grid = (B // _IB,)
    return pl.pallas_call(
        _encoder_body,
        grid=grid,
        in_specs=[
            pl.BlockSpec((_IB, 4, _NPAD), lambda i: (i, 0, 0)),
            pl.BlockSpec((_NPAD, D), lambda i: (0, 0)),
            pl.BlockSpec((_NLEV, D), lambda i: (0, 0)),
        ],
        out_specs=pl.BlockSpec((_IB, D), lambda i: (i, 0)),
        out_shape=jax.ShapeDtypeStruct((B, D), jnp.float32),
        scratch_shapes=[pltpu.VMEM((_IB * _NPAD, _NLEV), jnp.bfloat16)],
    )(x_pj, pw_pad, lw_bf16)
